# baseline (speedup 1.0000x reference)
import functools

import jax
import jax.numpy as jnp
from jax import lax
from jax.experimental import pallas as pl
from jax.experimental.pallas import tpu as pltpu
from jax.experimental.pallas import tpu_sc as plsc

NUM_LABELS = 1000000
EMBED_DIM = 32
BATCH = 16384
HIST = 50
TOTAL = BATCH * HIST

_NC = 2
_NS = 16
_NW = _NC * _NS
_PER_W = TOTAL // _NW
_CH = 128
_NCHUNK = _PER_W // _CH
_LANES = 128
_BLOCKS = NUM_LABELS * EMBED_DIM // _LANES


def _make_kernel():
  mesh = plsc.VectorSubcoreMesh(core_axis_name="c", subcore_axis_name="s")

  @functools.partial(
      pl.kernel,
      out_type=jax.ShapeDtypeStruct((_NW, _NCHUNK, _CH, EMBED_DIM), jnp.float32),
      mesh=mesh,
      compiler_params=pltpu.CompilerParams(
          use_tc_tiling_on_sc=True, needs_layout_passes=False),
      scratch_types=[
          pltpu.VMEM((_NCHUNK, _CH), jnp.int32),
          pltpu.VMEM((_NCHUNK, _CH), jnp.int32),
          pltpu.VMEM((_CH, _LANES), jnp.float32),
          pltpu.VMEM((_CH, _LANES), jnp.float32),
          pltpu.VMEM((_CH, EMBED_DIM), jnp.float32),
          pltpu.VMEM((_CH, EMBED_DIM), jnp.float32),
          pltpu.SemaphoreType.DMA,
          pltpu.SemaphoreType.DMA,
      ],
  )
  def gather_kernel(idx4_hbm, off_hbm, table_hbm, out_hbm,
                    idx4_v, off_v, pad0, pad1, cmp0, cmp1, g0, g1):
    wid = lax.axis_index("s") * _NC + lax.axis_index("c")
    pltpu.sync_copy(idx4_hbm.at[wid], idx4_v)
    pltpu.sync_copy(off_hbm.at[wid], off_v)

    def extract(j, pad, cmp):
      for grp in range(_CH // 16):
        p16 = jnp.arange(16, dtype=jnp.int32) + (grp * 16)
        col16 = off_v.at[j][pl.ds(grp * 16, 16)]
        for w in range(EMBED_DIM):
          vals = plsc.load_gather(pad, [p16, col16 + w])
          plsc.store_scatter(cmp, [p16, jnp.full((16,), w, jnp.int32)], vals)

    pltpu.async_copy(table_hbm.at[idx4_v.at[0]], pad0, g0)

    def body(h, _):
      j = h * 2
      up1 = pltpu.async_copy(table_hbm.at[idx4_v.at[j + 1]], pad1, g1)
      pltpu.make_async_copy(table_hbm.at[idx4_v.at[j]], pad0, g0).wait()
      extract(j, pad0, cmp0)
      pltpu.sync_copy(cmp0, out_hbm.at[wid, j])

      @pl.when(j + 2 < _NCHUNK)
      def _():
        pltpu.async_copy(table_hbm.at[idx4_v.at[j + 2]], pad0, g0)

      up1.wait()
      extract(j + 1, pad1, cmp1)
      pltpu.sync_copy(cmp1, out_hbm.at[wid, j + 1])
      return 0

    lax.fori_loop(0, _NCHUNK // 2, body, 0)

  return gather_kernel


_gather = _make_kernel()


def kernel(label_ids, table):
  ids = label_ids.astype(jnp.int32).reshape(_NW, _NCHUNK, _CH)
  idx4 = ids >> 2
  off = (ids & 3) << 5
  table2 = table.reshape(_BLOCKS, _LANES)
  out = _gather(idx4, off, table2)
  return out.reshape(BATCH, HIST, EMBED_DIM)


# speedup vs baseline: 1.1037x; 1.1037x over previous
import functools

import jax
import jax.numpy as jnp
from jax import lax
from jax.experimental import pallas as pl
from jax.experimental.pallas import tpu as pltpu
from jax.experimental.pallas import tpu_sc as plsc

NUM_LABELS = 1000000
EMBED_DIM = 32
HALF = EMBED_DIM // 2
BATCH = 16384
HIST = 50

_NC = 2
_NS = 16
_NW = _NC * _NS
_ROWS_W = BATCH // _NW
_RC = 8
_CH = _RC * HIST
_NCHUNK = _ROWS_W // _RC


def _make_kernel():
  mesh = plsc.VectorSubcoreMesh(core_axis_name="c", subcore_axis_name="s")
  half_out = jax.ShapeDtypeStruct((BATCH, HIST, HALF), jnp.float32)

  @functools.partial(
      pl.kernel,
      out_type=(half_out, half_out),
      mesh=mesh,
      compiler_params=pltpu.CompilerParams(use_tc_tiling_on_sc=False),
      scratch_types=[
          pltpu.VMEM((_NCHUNK, _CH), jnp.int32),
          pltpu.VMEM((_CH, HALF), jnp.float32),
          pltpu.VMEM((_CH, HALF), jnp.float32),
          pltpu.VMEM((_CH, HALF), jnp.float32),
          pltpu.VMEM((_CH, HALF), jnp.float32),
          pltpu.SemaphoreType.DMA,
          pltpu.SemaphoreType.DMA,
          pltpu.SemaphoreType.DMA,
          pltpu.SemaphoreType.DMA,
      ],
  )
  def gather_kernel(idx_hbm, ta_hbm, tb_hbm, oa_hbm, ob_hbm,
                    idx_v, a0, a1, b0, b1, ga0, ga1, gb0, gb1):
    wid = lax.axis_index("s") * _NC + lax.axis_index("c")
    base = wid * _ROWS_W
    pltpu.sync_copy(idx_hbm.at[wid], idx_v)

    def fire(j, abuf, bbuf, sa, sb):
      pltpu.async_copy(ta_hbm.at[idx_v.at[j]], abuf, sa)
      pltpu.async_copy(tb_hbm.at[idx_v.at[j]], bbuf, sb)

    def wait(j, abuf, bbuf, sa, sb):
      pltpu.make_async_copy(ta_hbm.at[idx_v.at[j]], abuf, sa).wait()
      pltpu.make_async_copy(tb_hbm.at[idx_v.at[j]], bbuf, sb).wait()

    def drain(abuf, bbuf, row0):
      for r in range(_RC):
        pltpu.sync_copy(abuf.at[pl.ds(r * HIST, HIST)], oa_hbm.at[row0 + r])
        pltpu.sync_copy(bbuf.at[pl.ds(r * HIST, HIST)], ob_hbm.at[row0 + r])

    fire(0, a0, b0, ga0, gb0)

    def body(h, _):
      j = h * 2
      fire(j + 1, a1, b1, ga1, gb1)
      wait(j, a0, b0, ga0, gb0)
      drain(a0, b0, base + j * _RC)

      @pl.when(j + 2 < _NCHUNK)
      def _():
        fire(j + 2, a0, b0, ga0, gb0)

      wait(j + 1, a1, b1, ga1, gb1)
      drain(a1, b1, base + (j + 1) * _RC)
      return 0

    lax.fori_loop(0, _NCHUNK // 2, body, 0)

  return gather_kernel


_gather = _make_kernel()


def kernel(label_ids, table):
  idx = label_ids.astype(jnp.int32).reshape(_NW, _NCHUNK, _CH)
  oa, ob = _gather(idx, table[:, :HALF], table[:, HALF:])
  return jnp.concatenate([oa, ob], axis=2)


# speedup vs baseline: 2.2803x; 2.0661x over previous
import functools

import jax
import jax.numpy as jnp
from jax import lax
from jax.experimental import pallas as pl
from jax.experimental.pallas import tpu as pltpu
from jax.experimental.pallas import tpu_sc as plsc

NUM_LABELS = 1000000
EMBED_DIM = 32
BATCH = 16384
HIST = 50

_NC = 2
_NS = 16
_NW = _NC * _NS
_ROWS_W = BATCH // _NW
_RC = 16
_CH = _RC * HIST
_NCHUNK = _ROWS_W // _RC


def _make_kernel():
  mesh = plsc.VectorSubcoreMesh(core_axis_name="c", subcore_axis_name="s")

  @functools.partial(
      pl.kernel,
      out_type=jax.ShapeDtypeStruct((BATCH, HIST, EMBED_DIM), jnp.float32),
      mesh=mesh,
      compiler_params=pltpu.CompilerParams(use_tc_tiling_on_sc=False),
      scratch_types=[
          pltpu.VMEM((_NCHUNK, _CH), jnp.int32),
          pltpu.VMEM((_CH, EMBED_DIM), jnp.float32),
          pltpu.VMEM((_CH, EMBED_DIM), jnp.float32),
          pltpu.SemaphoreType.DMA,
          pltpu.SemaphoreType.DMA,
      ],
  )
  def gather_kernel(idx_hbm, table_hbm, out_hbm, idx_v, rows0, rows1, g0, g1):
    wid = lax.axis_index("s") * _NC + lax.axis_index("c")
    base = wid * _ROWS_W
    pltpu.sync_copy(idx_hbm.at[wid], idx_v)

    pltpu.async_copy(table_hbm.at[idx_v.at[0]], rows0, g0)

    def drain(buf, row0):
      for r in range(_RC):
        pltpu.sync_copy(buf.at[pl.ds(r * HIST, HIST)], out_hbm.at[row0 + r])

    def body(h, _):
      j = h * 2
      up1 = pltpu.async_copy(table_hbm.at[idx_v.at[j + 1]], rows1, g1)
      pltpu.make_async_copy(table_hbm.at[idx_v.at[j]], rows0, g0).wait()
      drain(rows0, base + j * _RC)

      @pl.when(j + 2 < _NCHUNK)
      def _():
        pltpu.async_copy(table_hbm.at[idx_v.at[j + 2]], rows0, g0)

      up1.wait()
      drain(rows1, base + (j + 1) * _RC)
      return 0

    lax.fori_loop(0, _NCHUNK // 2, body, 0)

  return gather_kernel


_gather = _make_kernel()


def kernel(label_ids, table):
  idx = label_ids.astype(jnp.int32).reshape(_NW, _NCHUNK, _CH)
  return _gather(idx, table)
